# baseline (speedup 1.0000x reference)
import functools

import jax
import jax.numpy as jnp
from jax import lax
from jax.experimental import pallas as pl
from jax.experimental.pallas import tpu as pltpu
from jax.experimental.pallas import tpu_sc as plsc

B = 16384
EMB = 64
HID = 128
NC = 2
NS = 16
NW = NC * NS
BPW = B // NW
CHUNK = 128
NCHUNK = BPW // CHUNK


def _gather_body(user_hbm, item_hbm, ut_hbm, it_hbm, ug_hbm, ig_hbm,
                 uidx_v, iidx_v, urows_v, irows_v, sem):
    wid = lax.axis_index("s") * NC + lax.axis_index("c")
    base = wid * BPW
    row0 = wid * NCHUNK
    pltpu.sync_copy(user_hbm.at[pl.ds(row0, NCHUNK)], uidx_v)
    pltpu.sync_copy(item_hbm.at[pl.ds(row0, NCHUNK)], iidx_v)
    copies = []
    for j in range(NCHUNK):
        copies.append(pltpu.async_copy(
            ut_hbm.at[uidx_v.at[j]], urows_v.at[pl.ds(j * CHUNK, CHUNK)], sem))
        copies.append(pltpu.async_copy(
            it_hbm.at[iidx_v.at[j]], irows_v.at[pl.ds(j * CHUNK, CHUNK)], sem))
    for c in copies:
        c.wait()
    pltpu.sync_copy(urows_v, ug_hbm.at[pl.ds(base, BPW)])
    pltpu.sync_copy(irows_v, ig_hbm.at[pl.ds(base, BPW)])


@jax.jit
def _gather(user2d, item2d, user_table, item_table):
    mesh = plsc.VectorSubcoreMesh(core_axis_name="c", subcore_axis_name="s")
    f = functools.partial(
        pl.kernel,
        mesh=mesh,
        out_type=[
            jax.ShapeDtypeStruct((B, EMB), jnp.float32),
            jax.ShapeDtypeStruct((B, EMB), jnp.float32),
        ],
        scratch_types=[
            pltpu.VMEM((NCHUNK, CHUNK), jnp.int32),
            pltpu.VMEM((NCHUNK, CHUNK), jnp.int32),
            pltpu.VMEM((BPW, EMB), jnp.float32),
            pltpu.VMEM((BPW, EMB), jnp.float32),
            pltpu.SemaphoreType.DMA,
        ],
        compiler_params=pltpu.CompilerParams(use_tc_tiling_on_sc=False),
    )(_gather_body)
    return f(user2d, item2d, user_table, item_table)


def _mlp_body(ug_ref, ig_ref, w1u_ref, w1i_ref, b1_ref, w2_ref, b2_ref, out_ref):
    x = (jnp.dot(ug_ref[...], w1u_ref[...], preferred_element_type=jnp.float32)
         + jnp.dot(ig_ref[...], w1i_ref[...], preferred_element_type=jnp.float32)
         + b1_ref[...])
    h = jnp.maximum(x, 0.0)
    out_ref[...] = jnp.sum(h * w2_ref[...], axis=1, keepdims=True) + b2_ref[...]


ROWS = 1024


@jax.jit
def _mlp(ug, ig, w1u, w1i, b1, w2, b2):
    grid = (B // ROWS,)
    return pl.pallas_call(
        _mlp_body,
        grid=grid,
        in_specs=[
            pl.BlockSpec((ROWS, EMB), lambda g: (g, 0)),
            pl.BlockSpec((ROWS, EMB), lambda g: (g, 0)),
            pl.BlockSpec((EMB, HID), lambda g: (0, 0)),
            pl.BlockSpec((EMB, HID), lambda g: (0, 0)),
            pl.BlockSpec((1, HID), lambda g: (0, 0)),
            pl.BlockSpec((1, HID), lambda g: (0, 0)),
            pl.BlockSpec((1, 1), lambda g: (0, 0)),
        ],
        out_specs=pl.BlockSpec((ROWS, 1), lambda g: (g, 0)),
        out_shape=jax.ShapeDtypeStruct((B, 1), jnp.float32),
    )(ug, ig, w1u, w1i, b1, w2, b2)


def kernel(user, item, user_table, item_table, W1, b1, W2, b2):
    user2d = user.astype(jnp.int32).reshape(NW * NCHUNK, CHUNK)
    item2d = item.astype(jnp.int32).reshape(NW * NCHUNK, CHUNK)
    ug, ig = _gather(user2d, item2d, user_table, item_table)
    w1u = W1[:, :EMB].T
    w1i = W1[:, EMB:].T
    out = _mlp(ug, ig, w1u, w1i, b1.reshape(1, HID), W2, b2.reshape(1, 1))
    return out.reshape(B)


# speedup vs baseline: 1.0018x; 1.0018x over previous
import functools

import jax
import jax.numpy as jnp
from jax import lax
from jax.experimental import pallas as pl
from jax.experimental.pallas import tpu as pltpu
from jax.experimental.pallas import tpu_sc as plsc

B = 16384
EMB = 64
PAIR = 2 * EMB
HID = 128
NC = 2
NS = 16
NW = NC * NS
BPW = B // NW
CHUNK = 128
NCHUNK = BPW // CHUNK


def _gather_body(uidx_hbm, iidx_hbm, ut_hbm, it_hbm, ou_hbm, oi_hbm,
                 uidx_v, iidx_v, rows_v, sem):
    wid = lax.axis_index("s") * NC + lax.axis_index("c")
    base = wid * BPW
    row0 = wid * NCHUNK
    pltpu.sync_copy(uidx_hbm.at[pl.ds(row0, NCHUNK)], uidx_v)
    pltpu.sync_copy(iidx_hbm.at[pl.ds(row0, NCHUNK)], iidx_v)
    copies = []
    for j in range(NCHUNK):
        copies.append(pltpu.async_copy(
            ut_hbm.at[uidx_v.at[j]], rows_v.at[pl.ds(j * CHUNK, CHUNK)], sem))
    for c in copies:
        c.wait()
    pltpu.sync_copy(rows_v, ou_hbm.at[pl.ds(base, BPW)])
    copies = []
    for j in range(NCHUNK):
        copies.append(pltpu.async_copy(
            it_hbm.at[iidx_v.at[j]], rows_v.at[pl.ds(j * CHUNK, CHUNK)], sem))
    for c in copies:
        c.wait()
    pltpu.sync_copy(rows_v, oi_hbm.at[pl.ds(base, BPW)])


@jax.jit
def _gather(uidx2d, iidx2d, ut_pair, it_pair):
    mesh = plsc.VectorSubcoreMesh(core_axis_name="c", subcore_axis_name="s")
    f = functools.partial(
        pl.kernel,
        mesh=mesh,
        out_type=[
            jax.ShapeDtypeStruct((B, PAIR), jnp.float32),
            jax.ShapeDtypeStruct((B, PAIR), jnp.float32),
        ],
        scratch_types=[
            pltpu.VMEM((NCHUNK, CHUNK), jnp.int32),
            pltpu.VMEM((NCHUNK, CHUNK), jnp.int32),
            pltpu.VMEM((BPW, PAIR), jnp.float32),
            pltpu.SemaphoreType.DMA,
        ],
    )(_gather_body)
    return f(uidx2d, iidx2d, ut_pair, it_pair)


def _mlp_body(ug_ref, ig_ref, up_ref, ip_ref, w1u_ref, w1i_ref, b1_ref,
              w2_ref, b2_ref, out_ref):
    ue = jnp.dot(ug_ref[:, :EMB], w1u_ref[...],
                 preferred_element_type=jnp.float32)
    uo = jnp.dot(ug_ref[:, EMB:], w1u_ref[...],
                 preferred_element_type=jnp.float32)
    hu = jnp.where(up_ref[...] > 0, uo, ue)
    ie = jnp.dot(ig_ref[:, :EMB], w1i_ref[...],
                 preferred_element_type=jnp.float32)
    io = jnp.dot(ig_ref[:, EMB:], w1i_ref[...],
                 preferred_element_type=jnp.float32)
    hi = jnp.where(ip_ref[...] > 0, io, ie)
    h = jnp.maximum(hu + hi + b1_ref[...], 0.0)
    out_ref[...] = jnp.sum(h * w2_ref[...], axis=1, keepdims=True) + b2_ref[...]


ROWS = 1024


@jax.jit
def _mlp(ug, ig, up, ip, w1u, w1i, b1, w2, b2):
    grid = (B // ROWS,)
    return pl.pallas_call(
        _mlp_body,
        grid=grid,
        in_specs=[
            pl.BlockSpec((ROWS, PAIR), lambda g: (g, 0)),
            pl.BlockSpec((ROWS, PAIR), lambda g: (g, 0)),
            pl.BlockSpec((ROWS, 1), lambda g: (g, 0)),
            pl.BlockSpec((ROWS, 1), lambda g: (g, 0)),
            pl.BlockSpec((EMB, HID), lambda g: (0, 0)),
            pl.BlockSpec((EMB, HID), lambda g: (0, 0)),
            pl.BlockSpec((1, HID), lambda g: (0, 0)),
            pl.BlockSpec((1, HID), lambda g: (0, 0)),
            pl.BlockSpec((1, 1), lambda g: (0, 0)),
        ],
        out_specs=pl.BlockSpec((ROWS, 1), lambda g: (g, 0)),
        out_shape=jax.ShapeDtypeStruct((B, 1), jnp.float32),
    )(ug, ig, up, ip, w1u, w1i, b1, w2, b2)


def kernel(user, item, user_table, item_table, W1, b1, W2, b2):
    user = user.astype(jnp.int32)
    item = item.astype(jnp.int32)
    uidx2d = (user >> 1).reshape(NW * NCHUNK, CHUNK)
    iidx2d = (item >> 1).reshape(NW * NCHUNK, CHUNK)
    ut_pair = user_table.reshape(user_table.shape[0] // 2, PAIR)
    it_pair = item_table.reshape(item_table.shape[0] // 2, PAIR)
    ug, ig = _gather(uidx2d, iidx2d, ut_pair, it_pair)
    up = (user & 1).reshape(B, 1)
    ip = (item & 1).reshape(B, 1)
    w1u = W1[:, :EMB].T
    w1i = W1[:, EMB:].T
    out = _mlp(ug, ig, up, ip, w1u, w1i,
               b1.reshape(1, HID), W2, b2.reshape(1, 1))
    return out.reshape(B)


# speedup vs baseline: 1.1742x; 1.1721x over previous
import functools

import jax
import jax.numpy as jnp
from jax import lax
from jax.experimental import pallas as pl
from jax.experimental.pallas import tpu as pltpu
from jax.experimental.pallas import tpu_sc as plsc

B = 16384
EMB = 64
PAIR = 2 * EMB
HID = 128
TBL = 1000000
NC = 2
NS = 16
NW = NC * NS
BPW = B // NW
CHUNK = 128
NCHUNK = BPW // CHUNK

PACK_W = 2048
PACK_STEPS = -(-TBL // PACK_W)
PACK_ROWS = PACK_STEPS * PACK_W


def _pack_body(tt_ref, out_ref):
    xt = jnp.transpose(tt_ref[...], (1, 0))
    out_ref[...] = jnp.concatenate(
        [xt, jnp.zeros((PACK_W, EMB), jnp.float32)], axis=1)


@jax.jit
def _pack(tt):
    return pl.pallas_call(
        _pack_body,
        grid=(PACK_STEPS,),
        in_specs=[pl.BlockSpec((EMB, PACK_W), lambda g: (0, g))],
        out_specs=pl.BlockSpec((PACK_W, PAIR), lambda g: (g, 0)),
        out_shape=jax.ShapeDtypeStruct((PACK_ROWS, PAIR), jnp.float32),
    )(tt)


def _gather_body(idx_hbm, tab_hbm, out_hbm, idx_v, rows_v, sem):
    wid = lax.axis_index("s") * NC + lax.axis_index("c")
    pltpu.sync_copy(idx_hbm.at[pl.ds(wid * NCHUNK, NCHUNK)], idx_v)
    copies = []
    for j in range(NCHUNK):
        copies.append(pltpu.async_copy(
            tab_hbm.at[idx_v.at[j]], rows_v.at[pl.ds(j * CHUNK, CHUNK)], sem))
    for c in copies:
        c.wait()
    pltpu.sync_copy(rows_v, out_hbm.at[pl.ds(wid * BPW, BPW)])


@jax.jit
def _gather(idx2d, pack):
    mesh = plsc.VectorSubcoreMesh(core_axis_name="c", subcore_axis_name="s")
    f = functools.partial(
        pl.kernel,
        mesh=mesh,
        out_type=jax.ShapeDtypeStruct((B, PAIR), jnp.float32),
        scratch_types=[
            pltpu.VMEM((NCHUNK, CHUNK), jnp.int32),
            pltpu.VMEM((BPW, PAIR), jnp.float32),
            pltpu.SemaphoreType.DMA,
        ],
    )(_gather_body)
    return f(idx2d, pack)


def _mlp_body(ug_ref, ig_ref, w1u_ref, w1i_ref, b1_ref,
              w2_ref, b2_ref, out_ref):
    hu = jnp.dot(ug_ref[:, :EMB], w1u_ref[...],
                 preferred_element_type=jnp.float32)
    hi = jnp.dot(ig_ref[:, :EMB], w1i_ref[...],
                 preferred_element_type=jnp.float32)
    h = jnp.maximum(hu + hi + b1_ref[...], 0.0)
    out_ref[...] = jnp.sum(h * w2_ref[...], axis=1, keepdims=True) + b2_ref[...]


ROWS = 1024


@jax.jit
def _mlp(ug, ig, w1u, w1i, b1, w2, b2):
    grid = (B // ROWS,)
    return pl.pallas_call(
        _mlp_body,
        grid=grid,
        in_specs=[
            pl.BlockSpec((ROWS, PAIR), lambda g: (g, 0)),
            pl.BlockSpec((ROWS, PAIR), lambda g: (g, 0)),
            pl.BlockSpec((EMB, HID), lambda g: (0, 0)),
            pl.BlockSpec((EMB, HID), lambda g: (0, 0)),
            pl.BlockSpec((1, HID), lambda g: (0, 0)),
            pl.BlockSpec((1, HID), lambda g: (0, 0)),
            pl.BlockSpec((1, 1), lambda g: (0, 0)),
        ],
        out_specs=pl.BlockSpec((ROWS, 1), lambda g: (g, 0)),
        out_shape=jax.ShapeDtypeStruct((B, 1), jnp.float32),
    )(ug, ig, w1u, w1i, b1, w2, b2)


def kernel(user, item, user_table, item_table, W1, b1, W2, b2):
    user = user.astype(jnp.int32)
    item = item.astype(jnp.int32)
    ut_pack = _pack(user_table.T)
    it_pack = _pack(item_table.T)
    ug = _gather(user.reshape(NW * NCHUNK, CHUNK), ut_pack)
    ig = _gather(item.reshape(NW * NCHUNK, CHUNK), it_pack)
    w1u = W1[:, :EMB].T
    w1i = W1[:, EMB:].T
    out = _mlp(ug, ig, w1u, w1i,
               b1.reshape(1, HID), W2, b2.reshape(1, 1))
    return out.reshape(B)
